# baseline (speedup 1.0000x reference)
import jax
import jax.numpy as jnp
from jax.experimental import pallas as pl

T = 100000
S = 150000
P = 50000
A = 300000
B = 100


def _sage(h_src, h_dst, src, dst, wn, ws, b, n_dst, act):
    m = jnp.take(h_src, src, axis=0)
    s = jax.ops.segment_sum(m, dst, num_segments=n_dst)
    cnt = jax.ops.segment_sum(jnp.ones((dst.shape[0],), jnp.float32), dst, num_segments=n_dst)
    h_neigh = s / jnp.maximum(cnt, 1.0)[:, None]
    out = h_dst @ ws + h_neigh @ wn + b
    return jnp.tanh(out) if act else out


def _readout_mean(data, gid, n_graphs):
    s = jax.ops.segment_sum(data, gid, num_segments=n_graphs)
    cnt = jax.ops.segment_sum(jnp.ones((gid.shape[0],), jnp.float32), gid, num_segments=n_graphs)
    return s / jnp.maximum(cnt, 1.0)[:, None]


def _mlp_kernel(x_ref, w1_ref, b1_ref, w2_ref, b2_ref, w3_ref, b3_ref, o_ref):
    h = jnp.tanh(x_ref[...] @ w1_ref[...] + b1_ref[...][None, :])
    h = jnp.tanh(h @ w2_ref[...] + b2_ref[...][None, :])
    o_ref[...] = h @ w3_ref[...] + b3_ref[...][None, :]


def kernel(triangle_type, segment_type, boundary, angle_type, light_cone_angle, n_angle_types, n_light_cone_angle, tri_gid, seg_gid, pt_gid, ang_gid, sit_src, sit_dst, shp_src, shp_dst, sba_src, sba_dst, aap_src, aap_dst, tca_src, tca_dst, wn_sit, ws_sit, b_sit, wn_shp, ws_shp, b_shp, wn_sba, ws_sba, b_sba, wn_aap1, ws_aap1, b_aap1, wn_tca1, ws_tca1, b_tca1, wn_aap2, ws_aap2, b_aap2, wn_tca2, ws_tca2, b_tca2, wn_aap3, ws_aap3, b_aap3, gW1, gb1, gW2, gb2, gW3, gb3):
    tri_feats = triangle_type.astype(jnp.float32)[:, None]
    seg_feats = jnp.concatenate([boundary[:, None], segment_type.astype(jnp.float32)[:, None]], axis=1)
    angle_feats = jnp.concatenate([angle_type, light_cone_angle[:, None]], axis=1)
    pt_feats = jnp.concatenate([n_angle_types, n_light_cone_angle[:, None]], axis=1)
    cnt_t = jax.ops.segment_sum(jnp.ones((T,), jnp.float32), tri_gid, num_segments=B)
    cnt_s = jax.ops.segment_sum(jnp.ones((S,), jnp.float32), seg_gid, num_segments=B)
    cnt_p = jax.ops.segment_sum(jnp.ones((P,), jnp.float32), pt_gid, num_segments=B)
    log_n_tri = jnp.log(jnp.maximum(cnt_t, 1.0))[:, None]
    log_n_seg = jnp.log(jnp.maximum(cnt_s, 1.0))[:, None]
    log_n_pt = jnp.log(jnp.maximum(cnt_p, 1.0))[:, None]
    enc_tri = jax.nn.one_hot(triangle_type, 2, dtype=jnp.float32)
    frac_triangle_types = _readout_mean(enc_tri, tri_gid, B)
    enc_seg = jax.nn.one_hot(segment_type, 2, dtype=jnp.float32)
    frac_segment_types = _readout_mean(enc_seg, seg_gid, B)
    boundary_segments = boundary[:, None]
    frac_boundary_segments = _readout_mean(boundary_segments, seg_gid, B)
    frac_valid_segments = _readout_mean(enc_seg * boundary_segments, seg_gid, B)
    mean_complete_light_cones = _readout_mean(n_light_cone_angle[:, None] / 4.0, pt_gid, B)
    mean_angle_types = _readout_mean(n_angle_types, pt_gid, B)
    global_features = jnp.concatenate([log_n_tri, log_n_seg, log_n_pt, frac_triangle_types, frac_segment_types, frac_boundary_segments, frac_valid_segments, mean_complete_light_cones, mean_angle_types], axis=1)
    h_tri = _sage(seg_feats, tri_feats, sit_src, sit_dst, wn_sit, ws_sit, b_sit, T, True)
    h_pt = 0.5 * (_sage(seg_feats, pt_feats, shp_src, shp_dst, wn_shp, ws_shp, b_shp, P, True) + _sage(angle_feats, pt_feats, aap_src, aap_dst, wn_aap1, ws_aap1, b_aap1, P, True))
    h_ang = 0.5 * (_sage(seg_feats, angle_feats, sba_src, sba_dst, wn_sba, ws_sba, b_sba, A, True) + _sage(tri_feats, angle_feats, tca_src, tca_dst, wn_tca1, ws_tca1, b_tca1, A, True))
    h_pt2 = _sage(h_ang, h_pt, aap_src, aap_dst, wn_aap2, ws_aap2, b_aap2, P, True)
    h_ang2 = _sage(h_tri, h_ang, tca_src, tca_dst, wn_tca2, ws_tca2, b_tca2, A, True)
    point_logits = _sage(h_ang2, h_pt2, aap_src, aap_dst, wn_aap3, ws_aap3, b_aap3, P, False)
    triangulation_logits = pl.pallas_call(
        _mlp_kernel,
        out_shape=jax.ShapeDtypeStruct((B, 7), jnp.float32),
    )(global_features, gW1, gb1, gW2, gb2, gW3, gb3)
    return (point_logits, triangulation_logits)


# speedup vs baseline: 1.0730x; 1.0730x over previous
import jax
import jax.numpy as jnp
from jax.experimental import pallas as pl

T = 100000
S = 150000
P = 50000
A = 300000
B = 100

NC = 2
ZR = 1024
BLK = 1024
F32 = jnp.float32


def _rup(n, m):
    return ((n + m - 1) // m) * m


T_ACC = _rup(T + 1, ZR)
P_ACC = _rup(P + 1, ZR)
A_ACC = _rup(A + 1, ZR)
B_ACC = ZR

GT = T_ACC // BLK
GP = P_ACC // BLK
GA = 293


def _xla_acc(rows, dst, n, n_acc):
    w = rows.shape[1]
    s = jax.ops.segment_sum(rows, dst, num_segments=n)
    return jnp.zeros((NC * n_acc, w), F32).at[:n].set(s)


def _rows(wd):
    return pl.BlockSpec((BLK, wd), lambda i: (i, 0))


def _accspecs(wd, n_acc):
    off = n_acc // BLK
    return [pl.BlockSpec((BLK, wd), lambda i: (i, 0)),
            pl.BlockSpec((BLK, wd), lambda i, off=off: (i + off, 0))]


def _cst(shape):
    return pl.BlockSpec(shape, lambda i: (0,) * len(shape))


def _mean(a, k):
    return a[:, :k] / jnp.maximum(a[:, k:k + 1], 1.0)


def _tri_body(a0, a1, tf, ws, wn, b, wpre, o1, o2):
    a = a0[...] + a1[...]
    h = jnp.tanh(tf[...] @ ws[...] + _mean(a, 2) @ wn[...] + b[...][None, :])
    hp = h @ wpre[...]
    o1[...] = jnp.concatenate([hp[:, 0:4], jnp.ones((BLK, 1), F32)], axis=1)
    o2[...] = hp[:, 4:8]


def _pt_body(s0, s1, q0, q1, pf, ws1, wn1, b1, ws2, wn2, b2, wpre, o):
    m1 = _mean(s0[...] + s1[...], 2)
    m2 = _mean(q0[...] + q1[...], 5)
    x = pf[...]
    h = 0.5 * (jnp.tanh(x @ ws1[...] + m1 @ wn1[...] + b1[...][None, :])
               + jnp.tanh(x @ ws2[...] + m2 @ wn2[...] + b2[...][None, :]))
    o[...] = h @ wpre[...]


def _ang_body(s0, s1, q0, q1, af, ws1, wn1, b1, ws2, wn2, b2, wpre_n, wpre_s,
              o1, o2):
    m1 = _mean(s0[...] + s1[...], 2)
    m2 = _mean(q0[...] + q1[...], 1)
    x = af[...]
    h = 0.5 * (jnp.tanh(x @ ws1[...] + m1 @ wn1[...] + b1[...][None, :])
               + jnp.tanh(x @ ws2[...] + m2 @ wn2[...] + b2[...][None, :]))
    o1[...] = jnp.concatenate([h @ wpre_n[...], jnp.ones((BLK, 1), F32)],
                              axis=1)
    o2[...] = h @ wpre_s[...]


def _ang2_body(a0, a1, b0, b1, hs, bias, wpre, o):
    aa = a0[...] + a1[...]
    ab = b0[...] + b1[...]
    cnt = jnp.maximum(aa[:, 4:5], 1.0)
    mean = jnp.concatenate([aa[:, 0:4], ab[:, 0:4]], axis=1) / cnt
    h = jnp.tanh(hs[...] + mean + bias[...][None, :])
    o[...] = jnp.concatenate([h @ wpre[...], jnp.ones((BLK, 1), F32)], axis=1)


def _pt2_body(a0, a1, hs, bias, wpre, o):
    h = jnp.tanh(hs[...] + _mean(a0[...] + a1[...], 8) + bias[...][None, :])
    o[...] = h @ wpre[...]


def _logit_body(a0, a1, hs, bias, o):
    o[...] = hs[...] + _mean(a0[...] + a1[...], 1) + bias[...][None, :]


def _glob_body(t0, t1, s0, s1, p0, p1, w1, b1, w2, b2, w3, b3, o):
    at = t0[...] + t1[...]
    asg = s0[...] + s1[...]
    ap = p0[...] + p1[...]
    ct = jnp.maximum(at[:, 2:3], 1.0)
    cs = jnp.maximum(asg[:, 5:6], 1.0)
    cp = jnp.maximum(ap[:, 5:6], 1.0)
    gf = jnp.concatenate([
        jnp.log(ct), jnp.log(cs), jnp.log(cp),
        at[:, 0:2] / ct,
        asg[:, 0:2] / cs,
        asg[:, 2:3] / cs,
        asg[:, 3:5] / cs,
        ap[:, 0:1] / cp,
        ap[:, 1:5] / cp,
    ], axis=1)
    h = jnp.tanh(gf @ w1[...] + b1[...][None, :])
    h = jnp.tanh(h @ w2[...] + b2[...][None, :])
    o[...] = (h @ w3[...] + b3[...][None, :])[0:B]


def kernel(triangle_type, segment_type, boundary, angle_type,
           light_cone_angle, n_angle_types, n_light_cone_angle, tri_gid,
           seg_gid, pt_gid, ang_gid, sit_src, sit_dst, shp_src, shp_dst,
           sba_src, sba_dst, aap_src, aap_dst, tca_src, tca_dst, wn_sit,
           ws_sit, b_sit, wn_shp, ws_shp, b_shp, wn_sba, ws_sba, b_sba,
           wn_aap1, ws_aap1, b_aap1, wn_tca1, ws_tca1, b_tca1, wn_aap2,
           ws_aap2, b_aap2, wn_tca2, ws_tca2, b_tca2, wn_aap3, ws_aap3,
           b_aap3, gW1, gb1, gW2, gb2, gW3, gb3):
    ones_T = jnp.ones((T, 1), F32)
    ones_S = jnp.ones((S, 1), F32)
    ones_P = jnp.ones((P, 1), F32)
    ones_A = jnp.ones((A, 1), F32)
    tri_f = triangle_type.astype(F32)[:, None]
    seg_f = segment_type.astype(F32)[:, None]
    bnd = boundary[:, None]
    seg_tab = jnp.concatenate([bnd, seg_f, ones_S], axis=1)
    ang_tab = jnp.concatenate([angle_type, light_cone_angle[:, None], ones_A],
                              axis=1)
    tri_tab = jnp.concatenate([tri_f, ones_T], axis=1)
    angle_feats = ang_tab[:, 0:5]
    pt_feats = jnp.concatenate([n_angle_types, n_light_cone_angle[:, None]],
                               axis=1)
    enc_tri = jax.nn.one_hot(triangle_type, 2, dtype=F32)
    enc_seg = jax.nn.one_hot(segment_type, 2, dtype=F32)
    rt_tab = jnp.concatenate([enc_tri, ones_T], axis=1)
    rs_tab = jnp.concatenate([enc_seg, bnd, enc_seg * bnd, ones_S], axis=1)
    rp_tab = jnp.concatenate([n_light_cone_angle[:, None] / 4.0,
                              n_angle_types, ones_P], axis=1)

    acc_sit = _xla_acc(jnp.take(seg_tab, sit_src, axis=0), sit_dst, T, T_ACC)
    acc_shp = _xla_acc(jnp.take(seg_tab, shp_src, axis=0), shp_dst, P, P_ACC)
    acc_sba = _xla_acc(jnp.take(seg_tab, sba_src, axis=0), sba_dst, A, A_ACC)
    acc_aap1 = _xla_acc(jnp.take(ang_tab, aap_src, axis=0), aap_dst, P, P_ACC)
    acc_tca1 = _xla_acc(jnp.take(tri_tab, tca_src, axis=0), tca_dst, A, A_ACC)
    acc_rt = _xla_acc(rt_tab, tri_gid, B, B_ACC)
    acc_rs = _xla_acc(rs_tab, seg_gid, B, B_ACC)
    acc_rp = _xla_acc(rp_tab, pt_gid, B, B_ACC)

    tri2_a, tri2_b = pl.pallas_call(
        _tri_body,
        grid=(GT,),
        in_specs=_accspecs(3, T_ACC) + [_rows(1), _cst((1, 16)),
                                        _cst((2, 16)), _cst((16,)),
                                        _cst((16, 8))],
        out_specs=[_rows(5), _rows(4)],
        out_shape=[jax.ShapeDtypeStruct((T, 5), F32),
                   jax.ShapeDtypeStruct((T, 4), F32)],
    )(acc_sit, acc_sit, tri_f, ws_sit, wn_sit, b_sit, wn_tca2)

    hpt_ws = pl.pallas_call(
        _pt_body,
        grid=(GP,),
        in_specs=_accspecs(3, P_ACC) + _accspecs(6, P_ACC)
        + [_rows(5), _cst((5, 16)), _cst((2, 16)), _cst((16,)),
           _cst((5, 16)), _cst((5, 16)), _cst((16,)), _cst((16, 8))],
        out_specs=_rows(8),
        out_shape=jax.ShapeDtypeStruct((P, 8), F32),
    )(acc_shp, acc_shp, acc_aap1, acc_aap1, pt_feats, ws_shp, wn_shp, b_shp,
      ws_aap1, wn_aap1, b_aap1, ws_aap2)

    ang_n, ang_s = pl.pallas_call(
        _ang_body,
        grid=(GA,),
        in_specs=_accspecs(3, A_ACC) + _accspecs(2, A_ACC)
        + [_rows(5), _cst((5, 16)), _cst((2, 16)), _cst((16,)),
           _cst((5, 16)), _cst((1, 16)), _cst((16,)), _cst((16, 8)),
           _cst((16, 8))],
        out_specs=[_rows(9), _rows(8)],
        out_shape=[jax.ShapeDtypeStruct((A, 9), F32),
                   jax.ShapeDtypeStruct((A, 8), F32)],
    )(acc_sba, acc_sba, acc_tca1, acc_tca1, angle_feats, ws_sba, wn_sba,
      b_sba, ws_tca1, wn_tca1, b_tca1, wn_aap2, ws_tca2)

    acc_aap2 = _xla_acc(jnp.take(ang_n, aap_src, axis=0), aap_dst, P, P_ACC)
    acc_t2a = _xla_acc(jnp.take(tri2_a, tca_src, axis=0), tca_dst, A, A_ACC)
    acc_t2b = _xla_acc(jnp.take(tri2_b, tca_src, axis=0), tca_dst, A, A_ACC)

    ang2_n = pl.pallas_call(
        _ang2_body,
        grid=(GA,),
        in_specs=_accspecs(5, A_ACC) + _accspecs(4, A_ACC)
        + [_rows(8), _cst((8,)), _cst((8, 1))],
        out_specs=_rows(2),
        out_shape=jax.ShapeDtypeStruct((A, 2), F32),
    )(acc_t2a, acc_t2a, acc_t2b, acc_t2b, ang_s, b_tca2, wn_aap3)

    hpt2_ws = pl.pallas_call(
        _pt2_body,
        grid=(GP,),
        in_specs=_accspecs(9, P_ACC) + [_rows(8), _cst((8,)), _cst((8, 1))],
        out_specs=_rows(1),
        out_shape=jax.ShapeDtypeStruct((P, 1), F32),
    )(acc_aap2, acc_aap2, hpt_ws, b_aap2, ws_aap3)

    acc_aap3 = _xla_acc(jnp.take(ang2_n, aap_src, axis=0), aap_dst, P, P_ACC)

    point_logits = pl.pallas_call(
        _logit_body,
        grid=(GP,),
        in_specs=_accspecs(2, P_ACC) + [_rows(1), _cst((1,))],
        out_specs=_rows(1),
        out_shape=jax.ShapeDtypeStruct((P, 1), F32),
    )(acc_aap3, acc_aap3, hpt2_ws, b_aap3)

    bspec = [pl.BlockSpec((B_ACC, 3), lambda i: (0, 0)),
             pl.BlockSpec((B_ACC, 3), lambda i: (1, 0)),
             pl.BlockSpec((B_ACC, 6), lambda i: (0, 0)),
             pl.BlockSpec((B_ACC, 6), lambda i: (1, 0)),
             pl.BlockSpec((B_ACC, 6), lambda i: (0, 0)),
             pl.BlockSpec((B_ACC, 6), lambda i: (1, 0))]
    triangulation_logits = pl.pallas_call(
        _glob_body,
        grid=(1,),
        in_specs=bspec + [_cst((15, 32)), _cst((32,)), _cst((32, 16)),
                          _cst((16,)), _cst((16, 7)), _cst((7,))],
        out_specs=pl.BlockSpec((B, 7), lambda i: (0, 0)),
        out_shape=jax.ShapeDtypeStruct((B, 7), F32),
    )(acc_rt, acc_rt, acc_rs, acc_rs, acc_rp, acc_rp, gW1, gb1, gW2, gb2,
      gW3, gb3)

    return (point_logits, triangulation_logits)


# speedup vs baseline: 1.0895x; 1.0153x over previous
import jax
import jax.numpy as jnp
from jax.experimental import pallas as pl

T = 100000
S = 150000
P = 50000
A = 300000
B = 100

NC = 2
ZR = 1024
BLK = 1024
F32 = jnp.float32


def _rup(n, m):
    return ((n + m - 1) // m) * m


T_ACC = _rup(T + 1, ZR)
P_ACC = _rup(P + 1, ZR)
A_ACC = _rup(A + 1, ZR)
B_ACC = ZR

GT = T_ACC // BLK
GP = P_ACC // BLK
GA = 293


def _xla_acc(rows, dst, n, n_acc):
    w = rows.shape[1]
    s = jax.ops.segment_sum(rows, dst, num_segments=n)
    return jnp.zeros((NC * n_acc, w), F32).at[:n].set(s)


def _rows(wd):
    return pl.BlockSpec((BLK, wd), lambda i: (i, 0))


def _accspecs(wd, n_acc):
    off = n_acc // BLK
    return [pl.BlockSpec((BLK, wd), lambda i: (i, 0)),
            pl.BlockSpec((BLK, wd), lambda i, off=off: (i + off, 0))]


def _cst(shape):
    return pl.BlockSpec(shape, lambda i: (0,) * len(shape))


def _mean(a, k):
    return a[:, :k] / jnp.maximum(a[:, k:k + 1], 1.0)


def _tri_body(a0, a1, tf, ws, wn, b, wpre, o):
    a = a0[...] + a1[...]
    h = jnp.tanh(tf[...] @ ws[...] + _mean(a, 2) @ wn[...] + b[...][None, :])
    o[...] = h @ wpre[...]


def _pt_body(s0, s1, q0, q1, pf, ws1, wn1, b1, ws2, wn2, b2, wpre, o):
    m1 = _mean(s0[...] + s1[...], 2)
    m2 = _mean(q0[...] + q1[...], 5)
    x = pf[...]
    h = 0.5 * (jnp.tanh(x @ ws1[...] + m1 @ wn1[...] + b1[...][None, :])
               + jnp.tanh(x @ ws2[...] + m2 @ wn2[...] + b2[...][None, :]))
    o[...] = h @ wpre[...]


def _ang_body(s0, s1, q0, q1, af, ws1, wn1, b1, ws2, wn2, b2, wpre_n, wpre_s,
              o1, o2):
    m1 = _mean(s0[...] + s1[...], 2)
    m2 = _mean(q0[...] + q1[...], 1)
    x = af[...]
    h = 0.5 * (jnp.tanh(x @ ws1[...] + m1 @ wn1[...] + b1[...][None, :])
               + jnp.tanh(x @ ws2[...] + m2 @ wn2[...] + b2[...][None, :]))
    o1[...] = h @ wpre_n[...]
    o2[...] = h @ wpre_s[...]


def _ang2_body(a0, a1, c0, c1, hs, bias, wpre, o):
    a = a0[...] + a1[...]
    cnt = jnp.maximum((c0[...] + c1[...])[:, 1:2], 1.0)
    h = jnp.tanh(hs[...] + a / cnt + bias[...][None, :])
    o[...] = h @ wpre[...]


def _pt2_body(a0, a1, c0, c1, hs, bias, wpre, o):
    cnt = jnp.maximum((c0[...] + c1[...])[:, 5:6], 1.0)
    h = jnp.tanh(hs[...] + (a0[...] + a1[...]) / cnt + bias[...][None, :])
    o[...] = h @ wpre[...]


def _logit_body(a0, a1, c0, c1, hs, bias, o):
    cnt = jnp.maximum((c0[...] + c1[...])[:, 5:6], 1.0)
    o[...] = hs[...] + (a0[...] + a1[...]) / cnt + bias[...][None, :]


def _glob_body(t0, t1, s0, s1, p0, p1, w1, b1, w2, b2, w3, b3, o):
    at = t0[...] + t1[...]
    asg = s0[...] + s1[...]
    ap = p0[...] + p1[...]
    ct = jnp.maximum(at[:, 2:3], 1.0)
    cs = jnp.maximum(asg[:, 5:6], 1.0)
    cp = jnp.maximum(ap[:, 5:6], 1.0)
    gf = jnp.concatenate([
        jnp.log(ct), jnp.log(cs), jnp.log(cp),
        at[:, 0:2] / ct,
        asg[:, 0:2] / cs,
        asg[:, 2:3] / cs,
        asg[:, 3:5] / cs,
        ap[:, 0:1] / cp,
        ap[:, 1:5] / cp,
    ], axis=1)
    h = jnp.tanh(gf @ w1[...] + b1[...][None, :])
    h = jnp.tanh(h @ w2[...] + b2[...][None, :])
    o[...] = (h @ w3[...] + b3[...][None, :])[0:B]


def kernel(triangle_type, segment_type, boundary, angle_type,
           light_cone_angle, n_angle_types, n_light_cone_angle, tri_gid,
           seg_gid, pt_gid, ang_gid, sit_src, sit_dst, shp_src, shp_dst,
           sba_src, sba_dst, aap_src, aap_dst, tca_src, tca_dst, wn_sit,
           ws_sit, b_sit, wn_shp, ws_shp, b_shp, wn_sba, ws_sba, b_sba,
           wn_aap1, ws_aap1, b_aap1, wn_tca1, ws_tca1, b_tca1, wn_aap2,
           ws_aap2, b_aap2, wn_tca2, ws_tca2, b_tca2, wn_aap3, ws_aap3,
           b_aap3, gW1, gb1, gW2, gb2, gW3, gb3):
    ones_T = jnp.ones((T, 1), F32)
    ones_S = jnp.ones((S, 1), F32)
    ones_P = jnp.ones((P, 1), F32)
    ones_A = jnp.ones((A, 1), F32)
    tri_f = triangle_type.astype(F32)[:, None]
    seg_f = segment_type.astype(F32)[:, None]
    bnd = boundary[:, None]
    seg_tab = jnp.concatenate([bnd, seg_f, ones_S], axis=1)
    ang_tab = jnp.concatenate([angle_type, light_cone_angle[:, None], ones_A],
                              axis=1)
    tri_tab = jnp.concatenate([tri_f, ones_T], axis=1)
    angle_feats = ang_tab[:, 0:5]
    pt_feats = jnp.concatenate([n_angle_types, n_light_cone_angle[:, None]],
                               axis=1)
    enc_tri = jax.nn.one_hot(triangle_type, 2, dtype=F32)
    enc_seg = jax.nn.one_hot(segment_type, 2, dtype=F32)
    rt_tab = jnp.concatenate([enc_tri, ones_T], axis=1)
    rs_tab = jnp.concatenate([enc_seg, bnd, enc_seg * bnd, ones_S], axis=1)
    rp_tab = jnp.concatenate([n_light_cone_angle[:, None] / 4.0,
                              n_angle_types, ones_P], axis=1)

    acc_sit = _xla_acc(jnp.take(seg_tab, sit_src, axis=0), sit_dst, T, T_ACC)
    acc_shp = _xla_acc(jnp.take(seg_tab, shp_src, axis=0), shp_dst, P, P_ACC)
    acc_sba = _xla_acc(jnp.take(seg_tab, sba_src, axis=0), sba_dst, A, A_ACC)
    acc_aap1 = _xla_acc(jnp.take(ang_tab, aap_src, axis=0), aap_dst, P, P_ACC)
    acc_tca1 = _xla_acc(jnp.take(tri_tab, tca_src, axis=0), tca_dst, A, A_ACC)
    acc_rt = _xla_acc(rt_tab, tri_gid, B, B_ACC)
    acc_rs = _xla_acc(rs_tab, seg_gid, B, B_ACC)
    acc_rp = _xla_acc(rp_tab, pt_gid, B, B_ACC)

    tri2 = pl.pallas_call(
        _tri_body,
        grid=(GT,),
        in_specs=_accspecs(3, T_ACC) + [_rows(1), _cst((1, 16)),
                                        _cst((2, 16)), _cst((16,)),
                                        _cst((16, 8))],
        out_specs=_rows(8),
        out_shape=jax.ShapeDtypeStruct((T, 8), F32),
    )(acc_sit, acc_sit, tri_f, ws_sit, wn_sit, b_sit, wn_tca2)

    hpt_ws = pl.pallas_call(
        _pt_body,
        grid=(GP,),
        in_specs=_accspecs(3, P_ACC) + _accspecs(6, P_ACC)
        + [_rows(5), _cst((5, 16)), _cst((2, 16)), _cst((16,)),
           _cst((5, 16)), _cst((5, 16)), _cst((16,)), _cst((16, 8))],
        out_specs=_rows(8),
        out_shape=jax.ShapeDtypeStruct((P, 8), F32),
    )(acc_shp, acc_shp, acc_aap1, acc_aap1, pt_feats, ws_shp, wn_shp, b_shp,
      ws_aap1, wn_aap1, b_aap1, ws_aap2)

    ang_n, ang_s = pl.pallas_call(
        _ang_body,
        grid=(GA,),
        in_specs=_accspecs(3, A_ACC) + _accspecs(2, A_ACC)
        + [_rows(5), _cst((5, 16)), _cst((2, 16)), _cst((16,)),
           _cst((5, 16)), _cst((1, 16)), _cst((16,)), _cst((16, 8)),
           _cst((16, 8))],
        out_specs=[_rows(8), _rows(8)],
        out_shape=[jax.ShapeDtypeStruct((A, 8), F32),
                   jax.ShapeDtypeStruct((A, 8), F32)],
    )(acc_sba, acc_sba, acc_tca1, acc_tca1, angle_feats, ws_sba, wn_sba,
      b_sba, ws_tca1, wn_tca1, b_tca1, wn_aap2, ws_tca2)

    acc_aap2 = _xla_acc(jnp.take(ang_n, aap_src, axis=0), aap_dst, P, P_ACC)
    acc_t2 = _xla_acc(jnp.take(tri2, tca_src, axis=0), tca_dst, A, A_ACC)

    ang2_n = pl.pallas_call(
        _ang2_body,
        grid=(GA,),
        in_specs=_accspecs(8, A_ACC) + _accspecs(2, A_ACC)
        + [_rows(8), _cst((8,)), _cst((8, 1))],
        out_specs=_rows(1),
        out_shape=jax.ShapeDtypeStruct((A, 1), F32),
    )(acc_t2, acc_t2, acc_tca1, acc_tca1, ang_s, b_tca2, wn_aap3)

    hpt2_ws = pl.pallas_call(
        _pt2_body,
        grid=(GP,),
        in_specs=_accspecs(8, P_ACC) + _accspecs(6, P_ACC)
        + [_rows(8), _cst((8,)), _cst((8, 1))],
        out_specs=_rows(1),
        out_shape=jax.ShapeDtypeStruct((P, 1), F32),
    )(acc_aap2, acc_aap2, acc_aap1, acc_aap1, hpt_ws, b_aap2, ws_aap3)

    acc_aap3 = _xla_acc(jnp.take(ang2_n, aap_src, axis=0), aap_dst, P, P_ACC)

    point_logits = pl.pallas_call(
        _logit_body,
        grid=(GP,),
        in_specs=_accspecs(1, P_ACC) + _accspecs(6, P_ACC)
        + [_rows(1), _cst((1,))],
        out_specs=_rows(1),
        out_shape=jax.ShapeDtypeStruct((P, 1), F32),
    )(acc_aap3, acc_aap3, acc_aap1, acc_aap1, hpt2_ws, b_aap3)

    bspec = [pl.BlockSpec((B_ACC, 3), lambda i: (0, 0)),
             pl.BlockSpec((B_ACC, 3), lambda i: (1, 0)),
             pl.BlockSpec((B_ACC, 6), lambda i: (0, 0)),
             pl.BlockSpec((B_ACC, 6), lambda i: (1, 0)),
             pl.BlockSpec((B_ACC, 6), lambda i: (0, 0)),
             pl.BlockSpec((B_ACC, 6), lambda i: (1, 0))]
    triangulation_logits = pl.pallas_call(
        _glob_body,
        grid=(1,),
        in_specs=bspec + [_cst((15, 32)), _cst((32,)), _cst((32, 16)),
                          _cst((16,)), _cst((16, 7)), _cst((7,))],
        out_specs=pl.BlockSpec((B, 7), lambda i: (0, 0)),
        out_shape=jax.ShapeDtypeStruct((B, 7), F32),
    )(acc_rt, acc_rt, acc_rs, acc_rs, acc_rp, acc_rp, gW1, gb1, gW2, gb2,
      gW3, gb3)

    return (point_logits, triangulation_logits)


# speedup vs baseline: 1.1002x; 1.0099x over previous
import jax
import jax.numpy as jnp
from jax.experimental import pallas as pl

T = 100000
S = 150000
P = 50000
A = 300000
B = 100

NC = 2
ZR = 1024
BLK = 1024
F32 = jnp.float32


def _rup(n, m):
    return ((n + m - 1) // m) * m


T_ACC = _rup(T + 1, ZR)
P_ACC = _rup(P + 1, ZR)
A_ACC = _rup(A + 1, ZR)
B_ACC = ZR

GT = T_ACC // BLK
GP = P_ACC // BLK
GA = 293


def _xla_acc(rows, dst, n, n_acc, sorted_dst=False):
    w = rows.shape[1]
    s = jax.ops.segment_sum(rows, dst, num_segments=n,
                            indices_are_sorted=sorted_dst)
    return jnp.zeros((NC * n_acc, w), F32).at[:n].set(s)


def _rows(wd):
    return pl.BlockSpec((BLK, wd), lambda i: (i, 0))


def _accspecs(wd, n_acc):
    off = n_acc // BLK
    return [pl.BlockSpec((BLK, wd), lambda i: (i, 0)),
            pl.BlockSpec((BLK, wd), lambda i, off=off: (i + off, 0))]


def _cst(shape):
    return pl.BlockSpec(shape, lambda i: (0,) * len(shape))


def _mean(a, k):
    return a[:, :k] / jnp.maximum(a[:, k:k + 1], 1.0)


def _tri_body(a0, a1, tf, ws, wn, b, wpre, o):
    a = a0[...] + a1[...]
    h = jnp.tanh(tf[...] @ ws[...] + _mean(a, 2) @ wn[...] + b[...][None, :])
    o[...] = h @ wpre[...]


def _pt_body(s0, s1, q0, q1, pf, ws1, wn1, b1, ws2, wn2, b2, wpre, o):
    m1 = _mean(s0[...] + s1[...], 2)
    m2 = _mean(q0[...] + q1[...], 5)
    x = pf[...]
    h = 0.5 * (jnp.tanh(x @ ws1[...] + m1 @ wn1[...] + b1[...][None, :])
               + jnp.tanh(x @ ws2[...] + m2 @ wn2[...] + b2[...][None, :]))
    o[...] = h @ wpre[...]


def _ang_body(s0, s1, q0, q1, af, ws1, wn1, b1, ws2, wn2, b2, wpre_n, wpre_s,
              o1, o2):
    m1 = _mean(s0[...] + s1[...], 2)
    m2 = _mean(q0[...] + q1[...], 1)
    x = af[...]
    h = 0.5 * (jnp.tanh(x @ ws1[...] + m1 @ wn1[...] + b1[...][None, :])
               + jnp.tanh(x @ ws2[...] + m2 @ wn2[...] + b2[...][None, :]))
    o1[...] = h @ wpre_n[...]
    o2[...] = h @ wpre_s[...]


def _ang2_body(a0, a1, c0, c1, hs, bias, wpre, o):
    a = a0[...] + a1[...]
    cnt = jnp.maximum((c0[...] + c1[...])[:, 1:2], 1.0)
    h = jnp.tanh(hs[...] + a / cnt + bias[...][None, :])
    o[...] = h @ wpre[...]


def _pt2_body(a0, a1, c0, c1, hs, bias, wpre, o):
    cnt = jnp.maximum((c0[...] + c1[...])[:, 5:6], 1.0)
    h = jnp.tanh(hs[...] + (a0[...] + a1[...]) / cnt + bias[...][None, :])
    o[...] = h @ wpre[...]


def _logit_body(a0, a1, c0, c1, hs, bias, o):
    cnt = jnp.maximum((c0[...] + c1[...])[:, 5:6], 1.0)
    o[...] = hs[...] + (a0[...] + a1[...]) / cnt + bias[...][None, :]


def _glob_body(t0, t1, s0, s1, p0, p1, w1, b1, w2, b2, w3, b3, o):
    at = t0[...] + t1[...]
    asg = s0[...] + s1[...]
    ap = p0[...] + p1[...]
    ct = jnp.maximum(at[:, 2:3], 1.0)
    cs = jnp.maximum(asg[:, 5:6], 1.0)
    cp = jnp.maximum(ap[:, 5:6], 1.0)
    gf = jnp.concatenate([
        jnp.log(ct), jnp.log(cs), jnp.log(cp),
        at[:, 0:2] / ct,
        asg[:, 0:2] / cs,
        asg[:, 2:3] / cs,
        asg[:, 3:5] / cs,
        ap[:, 0:1] / cp,
        ap[:, 1:5] / cp,
    ], axis=1)
    h = jnp.tanh(gf @ w1[...] + b1[...][None, :])
    h = jnp.tanh(h @ w2[...] + b2[...][None, :])
    o[...] = (h @ w3[...] + b3[...][None, :])[0:B]


def kernel(triangle_type, segment_type, boundary, angle_type,
           light_cone_angle, n_angle_types, n_light_cone_angle, tri_gid,
           seg_gid, pt_gid, ang_gid, sit_src, sit_dst, shp_src, shp_dst,
           sba_src, sba_dst, aap_src, aap_dst, tca_src, tca_dst, wn_sit,
           ws_sit, b_sit, wn_shp, ws_shp, b_shp, wn_sba, ws_sba, b_sba,
           wn_aap1, ws_aap1, b_aap1, wn_tca1, ws_tca1, b_tca1, wn_aap2,
           ws_aap2, b_aap2, wn_tca2, ws_tca2, b_tca2, wn_aap3, ws_aap3,
           b_aap3, gW1, gb1, gW2, gb2, gW3, gb3):
    ones_T = jnp.ones((T, 1), F32)
    ones_S = jnp.ones((S, 1), F32)
    ones_P = jnp.ones((P, 1), F32)
    ones_A = jnp.ones((A, 1), F32)
    tri_f = triangle_type.astype(F32)[:, None]
    seg_f = segment_type.astype(F32)[:, None]
    bnd = boundary[:, None]
    seg_tab = jnp.concatenate([bnd, seg_f, ones_S], axis=1)
    ang_tab = jnp.concatenate([angle_type, light_cone_angle[:, None], ones_A],
                              axis=1)
    tri_tab = jnp.concatenate([tri_f, ones_T], axis=1)
    angle_feats = ang_tab[:, 0:5]
    pt_feats = jnp.concatenate([n_angle_types, n_light_cone_angle[:, None]],
                               axis=1)
    enc_tri = jax.nn.one_hot(triangle_type, 2, dtype=F32)
    enc_seg = jax.nn.one_hot(segment_type, 2, dtype=F32)
    rt_tab = jnp.concatenate([enc_tri, ones_T], axis=1)
    rs_tab = jnp.concatenate([enc_seg, bnd, enc_seg * bnd, ones_S], axis=1)
    rp_tab = jnp.concatenate([n_light_cone_angle[:, None] / 4.0,
                              n_angle_types, ones_P], axis=1)

    acc_sit = _xla_acc(jnp.take(seg_tab, sit_src, axis=0), sit_dst, T, T_ACC)
    acc_shp = _xla_acc(jnp.take(seg_tab, shp_src, axis=0), shp_dst, P, P_ACC)
    acc_sba = _xla_acc(jnp.take(seg_tab, sba_src, axis=0), sba_dst, A, A_ACC)
    acc_aap1 = _xla_acc(jnp.take(ang_tab, aap_src, axis=0), aap_dst, P, P_ACC)
    acc_tca1 = _xla_acc(jnp.take(tri_tab, tca_src, axis=0), tca_dst, A, A_ACC)
    acc_rt = _xla_acc(rt_tab, tri_gid, B, B_ACC, sorted_dst=True)
    acc_rs = _xla_acc(rs_tab, seg_gid, B, B_ACC, sorted_dst=True)
    acc_rp = _xla_acc(rp_tab, pt_gid, B, B_ACC, sorted_dst=True)

    tri2 = pl.pallas_call(
        _tri_body,
        grid=(GT,),
        in_specs=_accspecs(3, T_ACC) + [_rows(1), _cst((1, 16)),
                                        _cst((2, 16)), _cst((16,)),
                                        _cst((16, 8))],
        out_specs=_rows(8),
        out_shape=jax.ShapeDtypeStruct((T, 8), F32),
    )(acc_sit, acc_sit, tri_f, ws_sit, wn_sit, b_sit, wn_tca2)

    hpt_ws = pl.pallas_call(
        _pt_body,
        grid=(GP,),
        in_specs=_accspecs(3, P_ACC) + _accspecs(6, P_ACC)
        + [_rows(5), _cst((5, 16)), _cst((2, 16)), _cst((16,)),
           _cst((5, 16)), _cst((5, 16)), _cst((16,)), _cst((16, 8))],
        out_specs=_rows(8),
        out_shape=jax.ShapeDtypeStruct((P, 8), F32),
    )(acc_shp, acc_shp, acc_aap1, acc_aap1, pt_feats, ws_shp, wn_shp, b_shp,
      ws_aap1, wn_aap1, b_aap1, ws_aap2)

    ang_n, ang_s = pl.pallas_call(
        _ang_body,
        grid=(GA,),
        in_specs=_accspecs(3, A_ACC) + _accspecs(2, A_ACC)
        + [_rows(5), _cst((5, 16)), _cst((2, 16)), _cst((16,)),
           _cst((5, 16)), _cst((1, 16)), _cst((16,)), _cst((16, 8)),
           _cst((16, 8))],
        out_specs=[_rows(8), _rows(8)],
        out_shape=[jax.ShapeDtypeStruct((A, 8), F32),
                   jax.ShapeDtypeStruct((A, 8), F32)],
    )(acc_sba, acc_sba, acc_tca1, acc_tca1, angle_feats, ws_sba, wn_sba,
      b_sba, ws_tca1, wn_tca1, b_tca1, wn_aap2, ws_tca2)

    acc_aap2 = _xla_acc(jnp.take(ang_n, aap_src, axis=0), aap_dst, P, P_ACC)
    acc_t2 = _xla_acc(jnp.take(tri2, tca_src, axis=0), tca_dst, A, A_ACC)

    ang2_n = pl.pallas_call(
        _ang2_body,
        grid=(GA,),
        in_specs=_accspecs(8, A_ACC) + _accspecs(2, A_ACC)
        + [_rows(8), _cst((8,)), _cst((8, 1))],
        out_specs=_rows(1),
        out_shape=jax.ShapeDtypeStruct((A, 1), F32),
    )(acc_t2, acc_t2, acc_tca1, acc_tca1, ang_s, b_tca2, wn_aap3)

    hpt2_ws = pl.pallas_call(
        _pt2_body,
        grid=(GP,),
        in_specs=_accspecs(8, P_ACC) + _accspecs(6, P_ACC)
        + [_rows(8), _cst((8,)), _cst((8, 1))],
        out_specs=_rows(1),
        out_shape=jax.ShapeDtypeStruct((P, 1), F32),
    )(acc_aap2, acc_aap2, acc_aap1, acc_aap1, hpt_ws, b_aap2, ws_aap3)

    acc_aap3 = _xla_acc(jnp.take(ang2_n, aap_src, axis=0), aap_dst, P, P_ACC)

    point_logits = pl.pallas_call(
        _logit_body,
        grid=(GP,),
        in_specs=_accspecs(1, P_ACC) + _accspecs(6, P_ACC)
        + [_rows(1), _cst((1,))],
        out_specs=_rows(1),
        out_shape=jax.ShapeDtypeStruct((P, 1), F32),
    )(acc_aap3, acc_aap3, acc_aap1, acc_aap1, hpt2_ws, b_aap3)

    bspec = [pl.BlockSpec((B_ACC, 3), lambda i: (0, 0)),
             pl.BlockSpec((B_ACC, 3), lambda i: (1, 0)),
             pl.BlockSpec((B_ACC, 6), lambda i: (0, 0)),
             pl.BlockSpec((B_ACC, 6), lambda i: (1, 0)),
             pl.BlockSpec((B_ACC, 6), lambda i: (0, 0)),
             pl.BlockSpec((B_ACC, 6), lambda i: (1, 0))]
    triangulation_logits = pl.pallas_call(
        _glob_body,
        grid=(1,),
        in_specs=bspec + [_cst((15, 32)), _cst((32,)), _cst((32, 16)),
                          _cst((16,)), _cst((16, 7)), _cst((7,))],
        out_specs=pl.BlockSpec((B, 7), lambda i: (0, 0)),
        out_shape=jax.ShapeDtypeStruct((B, 7), F32),
    )(acc_rt, acc_rt, acc_rs, acc_rs, acc_rp, acc_rp, gW1, gb1, gW2, gb2,
      gW3, gb3)

    return (point_logits, triangulation_logits)
